# baseline (speedup 1.0000x reference)
import functools

import jax
import jax.numpy as jnp
from jax import lax
from jax.experimental import pallas as pl
from jax.experimental.pallas import tpu as pltpu
from jax.experimental.pallas import tpu_sc as plsc

B = 4096
D = 128
NLAB = 128
MARGIN = 0.2

NC = 2
NS = 16
NW = NC * NS
BPW = B // NW
L = 16

RB = 256
NSTEP = B // RB


def _select_body(t_ref, pos_ref, tneq_ref, pos2_ref, neg2_ref, nidx2_ref):
    BIG = jnp.int32(2**30)
    t = jnp.reshape(t_ref[...], (B, 1))
    T = jnp.broadcast_to(t, (B, NLAB))
    lab = lax.broadcasted_iota(jnp.int32, (B, NLAB), 1)
    ii = lax.broadcasted_iota(jnp.int32, (B, NLAB), 0)
    mask = T == lab
    f1 = jnp.min(jnp.where(mask, ii, BIG), axis=0, keepdims=True)
    mask2 = mask & (ii != f1)
    f2 = jnp.min(jnp.where(mask2, ii, BIG), axis=0, keepdims=True)
    f1_i = jnp.min(jnp.where(mask, f1, BIG), axis=1, keepdims=True)
    f2_i = jnp.min(jnp.where(mask, f2, BIG), axis=1, keepdims=True)
    icol = lax.broadcasted_iota(jnp.int32, (B, 1), 0)
    pos = jnp.where(f1_i != icol, f1_i, f2_i)
    pos = jnp.where(pos >= BIG, 0, pos)
    t0 = t_ref[0]
    g = jnp.min(jnp.where(T != t0, ii, BIG))
    g = jnp.where(g >= BIG, 0, g)
    tneq = (t != t0).astype(jnp.int32)
    neg = jnp.where(tneq != 0, 0, g)
    pos_ref[...] = pos
    tneq_ref[...] = tneq
    pos2_ref[...] = jnp.reshape(pos, (NW, BPW))
    neg2_ref[...] = jnp.reshape(neg, (NW, BPW))
    col8 = lax.broadcasted_iota(jnp.int32, (1, 8), 1)
    nidx2_ref[...] = jnp.where(col8 == 1, g, 0)


_select = pl.pallas_call(
    _select_body,
    out_shape=[
        jax.ShapeDtypeStruct((B, 1), jnp.int32),
        jax.ShapeDtypeStruct((B, 1), jnp.int32),
        jax.ShapeDtypeStruct((NW, BPW), jnp.int32),
        jax.ShapeDtypeStruct((NW, BPW), jnp.int32),
        jax.ShapeDtypeStruct((1, 8), jnp.int32),
    ],
)


MROW = RB // BPW


def _margins_for(dis_ref, pos, tneq, g):
    blk = dis_ref[...]
    cols = lax.broadcasted_iota(jnp.int32, (RB, B), 1)
    hp = jnp.sum(jnp.where(cols == pos, blk, 0.0), axis=1, keepdims=True)
    c0 = blk[:, 0:1]
    win = dis_ref[:, pl.ds((g // 128) * 128, 128)]
    wcol = lax.broadcasted_iota(jnp.int32, (RB, 128), 1)
    cg = jnp.sum(jnp.where(wcol == g % 128, win, 0.0), axis=1, keepdims=True)
    hn = jnp.where(tneq != 0, c0, cg)
    return jnp.maximum(hn - hp - MARGIN, 0.0) + MARGIN


NSTR = 4
_SSTEP = NSTEP // NSTR


def _extract_body(*refs):
    dis_refs = refs[:NSTR]
    pos_refs = refs[NSTR : 2 * NSTR]
    tneq_refs = refs[2 * NSTR : 3 * NSTR]
    nidx2_ref = refs[3 * NSTR]
    m_refs = refs[3 * NSTR + 1 :]
    g = nidx2_ref[0, 1]
    for k in range(NSTR):
        m_refs[k][...] = _margins_for(
            dis_refs[k], pos_refs[k][...], tneq_refs[k][...], g
        )


def _bspec(shape, stream):
    off = _SSTEP * stream
    return pl.BlockSpec(shape, lambda s, o=off: (s + o, 0))


_extract = pl.pallas_call(
    _extract_body,
    grid=(_SSTEP,),
    in_specs=(
        [_bspec((RB, B), k) for k in range(NSTR)]
        + [_bspec((RB, 1), k) for k in range(NSTR)]
        + [_bspec((RB, 1), k) for k in range(NSTR)]
        + [pl.BlockSpec(memory_space=pltpu.SMEM)]
    ),
    out_specs=[pl.BlockSpec((RB, 1), lambda s: (s, 0)) for _ in range(NSTR)],
    out_shape=[
        jax.ShapeDtypeStruct((B // NSTR, 1), jnp.float32) for _ in range(NSTR)
    ],
)


@functools.cache
def _build_sc_loss():
    mesh = plsc.VectorSubcoreMesh(
        core_axis_name="c", subcore_axis_name="s", num_cores=NC, num_subcores=NS
    )

    @functools.partial(
        pl.kernel,
        mesh=mesh,
        compiler_params=pltpu.CompilerParams(needs_layout_passes=False),
        out_type=jax.ShapeDtypeStruct((NW, BPW), jnp.float32),
        scratch_types=[
            pltpu.VMEM((BPW,), jnp.int32),
            pltpu.VMEM((BPW,), jnp.int32),
            pltpu.VMEM((2,), jnp.int32),
            pltpu.VMEM((BPW, D), jnp.float32),
            pltpu.VMEM((BPW, D), jnp.float32),
            pltpu.VMEM((2, D), jnp.float32),
            pltpu.VMEM((BPW,), jnp.float32),
            pltpu.SemaphoreType.DMA,
            pltpu.SemaphoreType.DMA,
            pltpu.SemaphoreType.DMA,
        ],
    )
    def _sc_loss(emb_hbm, pidx_hbm, nidx_hbm, nidx2_hbm, out_hbm,
                 pidx_v, nidx_v, nidx2_v, a_v, p_v, npair_v, d_v,
                 sem_a, sem_p, sem_n):
        wid = lax.axis_index("s") * NC + lax.axis_index("c")
        base = wid * BPW

        pltpu.sync_copy(pidx_hbm.at[wid], pidx_v)
        pltpu.sync_copy(nidx_hbm.at[wid], nidx_v)
        pltpu.sync_copy(nidx2_hbm.at[0, pl.ds(0, 2)], nidx2_v)

        cp_a = pltpu.async_copy(emb_hbm.at[pl.ds(base, BPW)], a_v, sem_a)
        cp_p = pltpu.async_copy(emb_hbm.at[pidx_v], p_v, sem_p)
        cp_n = pltpu.async_copy(emb_hbm.at[nidx2_v], npair_v, sem_n)
        cp_a.wait()
        cp_p.wait()
        cp_n.wait()

        lanes = lax.iota(jnp.int32, L)

        def group_body(gi, carry):
            rows = gi * L + lanes
            nsel = jnp.minimum(nidx_v[pl.ds(gi * L, L)], 1)
            accp = jnp.zeros((L,), jnp.float32)
            accn = jnp.zeros((L,), jnp.float32)
            for d in range(D):
                dcol = (lanes + d) & (D - 1)
                av = plsc.load_gather(a_v, [rows, dcol])
                pv = plsc.load_gather(p_v, [rows, dcol])
                nv = plsc.load_gather(npair_v, [nsel, dcol])
                dp = av - pv
                dn = av - nv
                accp = accp + dp * dp
                accn = accn + dn * dn
            d_v[pl.ds(gi * L, L)] = accp - accn
            return carry

        lax.fori_loop(0, BPW // L, group_body, jnp.int32(0))
        pltpu.sync_copy(d_v, out_hbm.at[wid])

    return _sc_loss


def _combine_body(d_ref, *refs):
    d = d_ref[...]
    m_refs, o_ref = refs[:NSTR], refs[NSTR]
    w = NW // NSTR
    total = jnp.float32(0.0)
    for k in range(NSTR):
        m = jnp.reshape(m_refs[k][...], (w, BPW))
        total += jnp.sum(jnp.maximum(d[k * w : (k + 1) * w] + m, 0.0))
    o_ref[...] = jnp.broadcast_to(total * (1.0 / B), (1, 1))


_combine = pl.pallas_call(
    _combine_body,
    out_shape=jax.ShapeDtypeStruct((1, 1), jnp.float32),
)


def kernel(embeddings, dis, target):
    t32 = target.astype(jnp.int32)
    pos, tneq, pos2, neg2, nidx2 = _select(t32)
    margins = _extract(
        *([dis] * NSTR), *([pos] * NSTR), *([tneq] * NSTR), nidx2
    )
    diffs = _build_sc_loss()(embeddings, pos2, neg2, nidx2)
    return _combine(diffs, *margins)[0, 0]


# speedup vs baseline: 1.0112x; 1.0112x over previous
import functools

import jax
import jax.numpy as jnp
from jax import lax
from jax.experimental import pallas as pl
from jax.experimental.pallas import tpu as pltpu
from jax.experimental.pallas import tpu_sc as plsc

B = 4096
D = 128
NLAB = 128
MARGIN = 0.2

NC = 2
NS = 16
NW = NC * NS
BPW = B // NW
L = 16

RB = 512
NSTEP = B // RB


def _select_body(t_ref, pos_ref, tneq_ref, pos2_ref, neg2_ref, nidx2_ref):
    BIG = jnp.int32(2**30)
    t = jnp.reshape(t_ref[...], (B, 1))
    T = jnp.broadcast_to(t, (B, NLAB))
    lab = lax.broadcasted_iota(jnp.int32, (B, NLAB), 1)
    ii = lax.broadcasted_iota(jnp.int32, (B, NLAB), 0)
    mask = T == lab
    f1 = jnp.min(jnp.where(mask, ii, BIG), axis=0, keepdims=True)
    mask2 = mask & (ii != f1)
    f2 = jnp.min(jnp.where(mask2, ii, BIG), axis=0, keepdims=True)
    f1_i = jnp.min(jnp.where(mask, f1, BIG), axis=1, keepdims=True)
    f2_i = jnp.min(jnp.where(mask, f2, BIG), axis=1, keepdims=True)
    icol = lax.broadcasted_iota(jnp.int32, (B, 1), 0)
    pos = jnp.where(f1_i != icol, f1_i, f2_i)
    pos = jnp.where(pos >= BIG, 0, pos)
    t0 = t_ref[0]
    g = jnp.min(jnp.where(T != t0, ii, BIG))
    g = jnp.where(g >= BIG, 0, g)
    tneq = (t != t0).astype(jnp.int32)
    neg = jnp.where(tneq != 0, 0, g)
    pos_ref[...] = pos
    tneq_ref[...] = tneq
    pos2_ref[...] = jnp.reshape(pos, (NW, BPW))
    neg2_ref[...] = jnp.reshape(neg, (NW, BPW))
    col8 = lax.broadcasted_iota(jnp.int32, (1, 8), 1)
    nidx2_ref[...] = jnp.where(col8 == 1, g, 0)


_select = pl.pallas_call(
    _select_body,
    out_shape=[
        jax.ShapeDtypeStruct((B, 1), jnp.int32),
        jax.ShapeDtypeStruct((B, 1), jnp.int32),
        jax.ShapeDtypeStruct((NW, BPW), jnp.int32),
        jax.ShapeDtypeStruct((NW, BPW), jnp.int32),
        jax.ShapeDtypeStruct((1, 8), jnp.int32),
    ],
)


MROW = RB // BPW


def _margins_for(dis_ref, pos, tneq, g):
    blk = dis_ref[...]
    cols = lax.broadcasted_iota(jnp.int32, (RB, B), 1)
    hp = jnp.sum(jnp.where(cols == pos, blk, 0.0), axis=1, keepdims=True)
    c0 = blk[:, 0:1]
    win = dis_ref[:, pl.ds((g // 128) * 128, 128)]
    wcol = lax.broadcasted_iota(jnp.int32, (RB, 128), 1)
    cg = jnp.sum(jnp.where(wcol == g % 128, win, 0.0), axis=1, keepdims=True)
    hn = jnp.where(tneq != 0, c0, cg)
    return jnp.maximum(hn - hp - MARGIN, 0.0) + MARGIN


NSTR = 2
_SSTEP = NSTEP // NSTR


def _extract_body(*refs):
    dis_refs = refs[:NSTR]
    pos_refs = refs[NSTR : 2 * NSTR]
    tneq_refs = refs[2 * NSTR : 3 * NSTR]
    nidx2_ref = refs[3 * NSTR]
    m_refs = refs[3 * NSTR + 1 :]
    g = nidx2_ref[0, 1]
    for k in range(NSTR):
        m_refs[k][...] = _margins_for(
            dis_refs[k], pos_refs[k][...], tneq_refs[k][...], g
        )


def _bspec(shape, stream):
    off = _SSTEP * stream
    return pl.BlockSpec(shape, lambda s, o=off: (s + o, 0))


_extract = pl.pallas_call(
    _extract_body,
    grid=(_SSTEP,),
    in_specs=(
        [_bspec((RB, B), k) for k in range(NSTR)]
        + [_bspec((RB, 1), k) for k in range(NSTR)]
        + [_bspec((RB, 1), k) for k in range(NSTR)]
        + [pl.BlockSpec(memory_space=pltpu.SMEM)]
    ),
    out_specs=[pl.BlockSpec((RB, 1), lambda s: (s, 0)) for _ in range(NSTR)],
    out_shape=[
        jax.ShapeDtypeStruct((B // NSTR, 1), jnp.float32) for _ in range(NSTR)
    ],
)


@functools.cache
def _build_sc_loss():
    mesh = plsc.VectorSubcoreMesh(
        core_axis_name="c", subcore_axis_name="s", num_cores=NC, num_subcores=NS
    )

    @functools.partial(
        pl.kernel,
        mesh=mesh,
        compiler_params=pltpu.CompilerParams(needs_layout_passes=False),
        out_type=jax.ShapeDtypeStruct((NW, BPW), jnp.float32),
        scratch_types=[
            pltpu.VMEM((BPW,), jnp.int32),
            pltpu.VMEM((BPW,), jnp.int32),
            pltpu.VMEM((2,), jnp.int32),
            pltpu.VMEM((BPW, D), jnp.float32),
            pltpu.VMEM((BPW, D), jnp.float32),
            pltpu.VMEM((2, D), jnp.float32),
            pltpu.VMEM((BPW,), jnp.float32),
            pltpu.SemaphoreType.DMA,
            pltpu.SemaphoreType.DMA,
            pltpu.SemaphoreType.DMA,
        ],
    )
    def _sc_loss(emb_hbm, pidx_hbm, nidx_hbm, nidx2_hbm, out_hbm,
                 pidx_v, nidx_v, nidx2_v, a_v, p_v, npair_v, d_v,
                 sem_a, sem_p, sem_n):
        wid = lax.axis_index("s") * NC + lax.axis_index("c")
        base = wid * BPW

        pltpu.sync_copy(pidx_hbm.at[wid], pidx_v)
        pltpu.sync_copy(nidx_hbm.at[wid], nidx_v)
        pltpu.sync_copy(nidx2_hbm.at[0, pl.ds(0, 2)], nidx2_v)

        cp_a = pltpu.async_copy(emb_hbm.at[pl.ds(base, BPW)], a_v, sem_a)
        cp_p = pltpu.async_copy(emb_hbm.at[pidx_v], p_v, sem_p)
        cp_n = pltpu.async_copy(emb_hbm.at[nidx2_v], npair_v, sem_n)
        cp_a.wait()
        cp_p.wait()
        cp_n.wait()

        lanes = lax.iota(jnp.int32, L)

        def group_body(gi, carry):
            rows = gi * L + lanes
            nsel = jnp.minimum(nidx_v[pl.ds(gi * L, L)], 1)
            accp = jnp.zeros((L,), jnp.float32)
            accn = jnp.zeros((L,), jnp.float32)
            for d in range(D):
                dcol = (lanes + d) & (D - 1)
                av = plsc.load_gather(a_v, [rows, dcol])
                pv = plsc.load_gather(p_v, [rows, dcol])
                nv = plsc.load_gather(npair_v, [nsel, dcol])
                dp = av - pv
                dn = av - nv
                accp = accp + dp * dp
                accn = accn + dn * dn
            d_v[pl.ds(gi * L, L)] = accp - accn
            return carry

        lax.fori_loop(0, BPW // L, group_body, jnp.int32(0))
        pltpu.sync_copy(d_v, out_hbm.at[wid])

    return _sc_loss


def _combine_body(d_ref, *refs):
    d = d_ref[...]
    m_refs, o_ref = refs[:NSTR], refs[NSTR]
    w = NW // NSTR
    total = jnp.float32(0.0)
    for k in range(NSTR):
        m = jnp.reshape(m_refs[k][...], (w, BPW))
        total += jnp.sum(jnp.maximum(d[k * w : (k + 1) * w] + m, 0.0))
    o_ref[...] = jnp.broadcast_to(total * (1.0 / B), (1, 1))


_combine = pl.pallas_call(
    _combine_body,
    out_shape=jax.ShapeDtypeStruct((1, 1), jnp.float32),
)


def kernel(embeddings, dis, target):
    t32 = target.astype(jnp.int32)
    pos, tneq, pos2, neg2, nidx2 = _select(t32)
    margins = _extract(
        *([dis] * NSTR), *([pos] * NSTR), *([tneq] * NSTR), nidx2
    )
    diffs = _build_sc_loss()(embeddings, pos2, neg2, nidx2)
    return _combine(diffs, *margins)[0, 0]
